# baseline (speedup 1.0000x reference)
import functools

import jax
import jax.numpy as jnp
from jax import lax
from jax.experimental import pallas as pl
from jax.experimental.pallas import tpu as pltpu
from jax.experimental.pallas import tpu_sc as plsc

B = 16384
E = 16
NC = 2
NS = 16
NW = NC * NS
BPW = B // NW
CH = 128
NCH = BPW // CH
V = 1000000
VA = 999936
NT = V - VA
PAD = 1000448


def _tc_linearize(u_tabT, r_tabT):
    def body(u_ref, r_ref, uo_ref, ro_ref, sem):
        copies = []
        for l in range(E):
            dst = pl.ds(l * PAD, VA)
            src = pl.ds(0, VA)
            copies.append(pltpu.make_async_copy(u_ref.at[l, src], uo_ref.at[dst], sem))
            copies.append(pltpu.make_async_copy(r_ref.at[l, src], ro_ref.at[dst], sem))
        for c in copies:
            c.start()
        for c in copies:
            c.wait()

    return pl.pallas_call(
        body,
        in_specs=[pl.BlockSpec(memory_space=pl.ANY)] * 2,
        out_specs=[pl.BlockSpec(memory_space=pl.ANY)] * 2,
        out_shape=[jax.ShapeDtypeStruct((E * PAD,), jnp.float32)] * 2,
        scratch_shapes=[pltpu.SemaphoreType.DMA],
    )(u_tabT, r_tabT)


def _sc_gather_dot(u_lidx, r_lidx, u_idx2d, r_idx2d, u_lin, r_lin,
                   u_tail, r_tail, u_bias, r_bias):
    mesh = plsc.VectorSubcoreMesh(core_axis_name="c", subcore_axis_name="s")

    @functools.partial(
        pl.kernel,
        mesh=mesh,
        out_type=(
            jax.ShapeDtypeStruct((NW * E,), jnp.float32),
            jax.ShapeDtypeStruct((B,), jnp.float32),
        ),
        scratch_types=[
            pltpu.VMEM((E, BPW), jnp.int32),
            pltpu.VMEM((E, BPW), jnp.int32),
            pltpu.VMEM((NCH, CH), jnp.int32),
            pltpu.VMEM((NCH, CH), jnp.int32),
            pltpu.VMEM((E, BPW), jnp.float32),
            pltpu.VMEM((E, BPW), jnp.float32),
            pltpu.VMEM((NT, E), jnp.float32),
            pltpu.VMEM((NT, E), jnp.float32),
            pltpu.VMEM((BPW,), jnp.float32),
            pltpu.VMEM((BPW,), jnp.float32),
            pltpu.VMEM((BPW,), jnp.float32),
            pltpu.VMEM((E,), jnp.float32),
            pltpu.SemaphoreType.DMA,
        ],
        compiler_params=pltpu.CompilerParams(
            use_tc_tiling_on_sc=False, needs_layout_passes=False),
    )
    def k(u_lidx_hbm, r_lidx_hbm, u_idx_hbm, r_idx_hbm, u_lin_hbm, r_lin_hbm,
          u_tail_hbm, r_tail_hbm, u_bias_hbm, r_bias_hbm, partial_hbm, ubrb_hbm,
          lidx_u, lidx_r, idx_u, idx_r, u_vals, r_vals, ut_v, rt_v, ub_v, rb_v,
          ubrb_v, acc_v, sem):
        wid = lax.axis_index("s") * NC + lax.axis_index("c")
        base = pl.multiple_of(wid * BPW, 8)
        row0 = wid * NCH

        pltpu.sync_copy(u_lidx_hbm.at[wid], lidx_u)
        pltpu.sync_copy(r_lidx_hbm.at[wid], lidx_r)
        pltpu.sync_copy(u_idx_hbm.at[pl.ds(row0, NCH)], idx_u)
        pltpu.sync_copy(r_idx_hbm.at[pl.ds(row0, NCH)], idx_r)
        pltpu.sync_copy(u_tail_hbm, ut_v)
        pltpu.sync_copy(r_tail_hbm, rt_v)

        for j in range(NCH):
            sl = pl.ds(j * CH, CH)
            copies = []
            for l in range(E):
                copies.append(pltpu.async_copy(
                    u_lin_hbm.at[lidx_u.at[l, sl]], u_vals.at[l, sl], sem))
                copies.append(pltpu.async_copy(
                    r_lin_hbm.at[lidx_r.at[l, sl]], r_vals.at[l, sl], sem))
            copies.append(pltpu.async_copy(u_bias_hbm.at[idx_u.at[j]], ub_v.at[sl], sem))
            copies.append(pltpu.async_copy(r_bias_hbm.at[idx_r.at[j]], rb_v.at[sl], sem))
            for c in copies:
                c.wait()

        acc = jnp.zeros((E,), jnp.float32)
        for j in range(NCH):
            for i8 in range(CH // E):
                g = pl.ds(j * CH + i8 * E, E)
                gi = pl.ds(i8 * E, E)
                iu = idx_u[j, gi]
                ir = idx_r[j, gi]
                mu = iu >= VA
                mr = ir >= VA
                tu = jnp.maximum(iu - VA, 0)
                tr = jnp.maximum(ir - VA, 0)
                for l in range(E):
                    lcol = jnp.full((E,), l, jnp.int32)
                    uv = jnp.where(mu, plsc.load_gather(ut_v, [tu, lcol]),
                                   u_vals[l, g])
                    rv = jnp.where(mr, plsc.load_gather(rt_v, [tr, lcol]),
                                   r_vals[l, g])
                    acc = acc + uv * rv
        acc_v[...] = acc
        pltpu.sync_copy(acc_v, partial_hbm.at[pl.ds(pl.multiple_of(wid * E, 8), E)])

        for i in range(BPW // E):
            sl = pl.ds(i * E, E)
            ubrb_v[sl] = ub_v[sl] + rb_v[sl]
        pltpu.sync_copy(ubrb_v, ubrb_hbm.at[pl.ds(base, BPW)])

    return k(u_lidx, r_lidx, u_idx2d, r_idx2d, u_lin, r_lin,
             u_tail, r_tail, u_bias, r_bias)


def _tc_finish(partials_2d, ubrb_2d):
    def body(p_ref, x_ref, o_ref):
        s = jnp.sum(p_ref[...])
        o_ref[...] = jax.nn.sigmoid(x_ref[...] + s)

    return pl.pallas_call(
        body,
        out_shape=jax.ShapeDtypeStruct(ubrb_2d.shape, jnp.float32),
    )(partials_2d, ubrb_2d)


def kernel(inputs, user_embedding, user_bias, resto_embedding, resto_bias):
    idx = inputs.astype(jnp.int32)
    u_idx = idx[:, 0]
    r_idx = idx[:, 1]
    lane_off = (jnp.arange(E, dtype=jnp.int32) * PAD).reshape(1, E, 1)
    u_lidx = jnp.minimum(u_idx, VA - 1).reshape(NW, 1, BPW) + lane_off
    r_lidx = jnp.minimum(r_idx, VA - 1).reshape(NW, 1, BPW) + lane_off
    u_lin, r_lin = _tc_linearize(user_embedding.T, resto_embedding.T)
    partials, ubrb = _sc_gather_dot(
        u_lidx, r_lidx,
        u_idx.reshape(B // CH, CH), r_idx.reshape(B // CH, CH),
        u_lin, r_lin,
        user_embedding[VA:, :], resto_embedding[VA:, :],
        user_bias.reshape(-1), resto_bias.reshape(-1))
    out = _tc_finish(partials.reshape(NW * E // 128, 128), ubrb.reshape(B // 128, 128))
    return out.reshape(B, 1)


# speedup vs baseline: 4.9904x; 4.9904x over previous
import functools

import jax
import jax.numpy as jnp
from jax import lax
from jax.experimental import pallas as pl
from jax.experimental.pallas import tpu as pltpu
from jax.experimental.pallas import tpu_sc as plsc

B = 16384
E = 16
NC = 2
NS = 16
NW = NC * NS
BPW = B // NW
CH = 128
NCH = BPW // CH


def _sc_gather_dot(u_idx2d, r_idx2d, u_emb, r_emb, u_bias, r_bias):
    mesh = plsc.VectorSubcoreMesh(core_axis_name="c", subcore_axis_name="s")

    @functools.partial(
        pl.kernel,
        mesh=mesh,
        out_type=(
            jax.ShapeDtypeStruct((NW * E,), jnp.float32),
            jax.ShapeDtypeStruct((B,), jnp.float32),
        ),
        scratch_types=[
            pltpu.VMEM((NCH, CH), jnp.int32),
            pltpu.VMEM((NCH, CH), jnp.int32),
            pltpu.VMEM((BPW, E), jnp.float32),
            pltpu.VMEM((BPW, E), jnp.float32),
            pltpu.VMEM((BPW,), jnp.float32),
            pltpu.VMEM((BPW,), jnp.float32),
            pltpu.VMEM((BPW,), jnp.float32),
            pltpu.VMEM((E,), jnp.float32),
            pltpu.SemaphoreType.DMA,
        ],
        compiler_params=pltpu.CompilerParams(use_tc_tiling_on_sc=False),
    )
    def k(u_idx_hbm, r_idx_hbm, u_emb_hbm, r_emb_hbm, u_bias_hbm, r_bias_hbm,
          partial_hbm, ubrb_hbm, idx_u, idx_r, u_rows, r_rows, ub_v, rb_v,
          ubrb_v, acc_v, sem):
        wid = lax.axis_index("s") * NC + lax.axis_index("c")
        base = pl.multiple_of(wid * BPW, 8)
        row0 = wid * NCH

        pltpu.sync_copy(u_idx_hbm.at[pl.ds(row0, NCH)], idx_u)
        pltpu.sync_copy(r_idx_hbm.at[pl.ds(row0, NCH)], idx_r)

        copies = []
        for j in range(NCH):
            sl = pl.ds(j * CH, CH)
            copies.append(pltpu.async_copy(u_emb_hbm.at[idx_u.at[j]], u_rows.at[sl], sem))
            copies.append(pltpu.async_copy(r_emb_hbm.at[idx_r.at[j]], r_rows.at[sl], sem))
            copies.append(pltpu.async_copy(u_bias_hbm.at[idx_u.at[j]], ub_v.at[sl], sem))
            copies.append(pltpu.async_copy(r_bias_hbm.at[idx_r.at[j]], rb_v.at[sl], sem))
        for c in copies:
            c.wait()

        def dot_body(i, acc):
            return acc + u_rows[i, :] * r_rows[i, :]

        acc_v[...] = lax.fori_loop(0, BPW, dot_body, jnp.zeros((E,), jnp.float32))
        pltpu.sync_copy(acc_v, partial_hbm.at[pl.ds(pl.multiple_of(wid * E, 8), E)])

        for i in range(BPW // E):
            sl = pl.ds(i * E, E)
            ubrb_v[sl] = ub_v[sl] + rb_v[sl]
        pltpu.sync_copy(ubrb_v, ubrb_hbm.at[pl.ds(base, BPW)])

    return k(u_idx2d, r_idx2d, u_emb, r_emb, u_bias, r_bias)


def _tc_finish(partials_2d, ubrb_2d):
    def body(p_ref, x_ref, o_ref):
        s = jnp.sum(p_ref[...])
        o_ref[...] = jax.nn.sigmoid(x_ref[...] + s)

    return pl.pallas_call(
        body,
        out_shape=jax.ShapeDtypeStruct(ubrb_2d.shape, jnp.float32),
    )(partials_2d, ubrb_2d)


def kernel(inputs, user_embedding, user_bias, resto_embedding, resto_bias):
    idx = inputs.astype(jnp.int32)
    u_idx2d = idx[:, 0].reshape(B // CH, CH)
    r_idx2d = idx[:, 1].reshape(B // CH, CH)
    partials, ubrb = _sc_gather_dot(
        u_idx2d, r_idx2d, user_embedding, resto_embedding,
        user_bias.reshape(-1), resto_bias.reshape(-1))
    out = _tc_finish(partials.reshape(NW * E // 128, 128), ubrb.reshape(B // 128, 128))
    return out.reshape(B, 1)


# speedup vs baseline: 24.5266x; 4.9147x over previous
import functools

import jax
import jax.numpy as jnp
from jax import lax
from jax.experimental import pallas as pl
from jax.experimental.pallas import tpu as pltpu
from jax.experimental.pallas import tpu_sc as plsc

B = 16384
E = 16
NC = 2
NS = 16
NW = NC * NS
BPW = B // NW
CH = 128
NCH = BPW // CH
V = 1000000
VA = 999936
NT = V - VA
PAD = 1000448
CHW = 32256
NLCH = VA // CHW


def _sc_linearize(u_tabT, r_tabT):
    mesh = plsc.VectorSubcoreMesh(core_axis_name="c", subcore_axis_name="s")

    @functools.partial(
        pl.kernel,
        mesh=mesh,
        out_type=(
            jax.ShapeDtypeStruct((E * PAD,), jnp.float32),
            jax.ShapeDtypeStruct((E * PAD,), jnp.float32),
        ),
        scratch_types=[
            pltpu.VMEM((2, CHW), jnp.float32),
            pltpu.SemaphoreType.DMA,
            pltpu.SemaphoreType.DMA,
            pltpu.SemaphoreType.DMA,
        ],
        compiler_params=pltpu.CompilerParams(needs_layout_passes=False),
    )
    def k(u_tab_hbm, r_tab_hbm, u_out_hbm, r_out_hbm, buf, sem_in, s_w0, s_w1):
        wid = lax.axis_index("s") * NC + lax.axis_index("c")
        lane = lax.rem(wid, E)
        wsems = (s_w0, s_w1)

        def do_table(tab, out):
            reads = [None, None]
            writes = [None, None]
            reads[0] = pltpu.async_copy(
                tab.at[lane, pl.ds(0, CHW)], buf.at[0], sem_in)
            for c in range(NLCH):
                b = c % 2
                reads[b].wait()
                if c + 1 < NLCH:
                    nb = (c + 1) % 2
                    if writes[nb] is not None:
                        writes[nb].wait()
                    reads[nb] = pltpu.async_copy(
                        tab.at[lane, pl.ds((c + 1) * CHW, CHW)], buf.at[nb], sem_in)
                writes[b] = pltpu.async_copy(
                    buf.at[b], out.at[pl.ds(lane * PAD + c * CHW, CHW)], wsems[b])
            for w in writes:
                if w is not None:
                    w.wait()

        @pl.when(wid < E)
        def _():
            do_table(u_tab_hbm, u_out_hbm)

        @pl.when(wid >= E)
        def _():
            do_table(r_tab_hbm, r_out_hbm)

    return k(u_tabT, r_tabT)


def _sc_gather_dot(u_lidx, r_lidx, u_idx2d, r_idx2d, u_lin, r_lin,
                   u_tail, r_tail, u_bias, r_bias):
    mesh = plsc.VectorSubcoreMesh(core_axis_name="c", subcore_axis_name="s")

    @functools.partial(
        pl.kernel,
        mesh=mesh,
        out_type=(
            jax.ShapeDtypeStruct((NW * E,), jnp.float32),
            jax.ShapeDtypeStruct((B,), jnp.float32),
        ),
        scratch_types=[
            pltpu.VMEM((E, BPW), jnp.int32),
            pltpu.VMEM((E, BPW), jnp.int32),
            pltpu.VMEM((NCH, CH), jnp.int32),
            pltpu.VMEM((NCH, CH), jnp.int32),
            pltpu.VMEM((E, BPW), jnp.float32),
            pltpu.VMEM((E, BPW), jnp.float32),
            pltpu.VMEM((NT, E), jnp.float32),
            pltpu.VMEM((NT, E), jnp.float32),
            pltpu.VMEM((BPW,), jnp.float32),
            pltpu.VMEM((BPW,), jnp.float32),
            pltpu.VMEM((BPW,), jnp.float32),
            pltpu.VMEM((E,), jnp.float32),
            pltpu.SemaphoreType.DMA,
        ],
        compiler_params=pltpu.CompilerParams(
            use_tc_tiling_on_sc=False, needs_layout_passes=False),
    )
    def k(u_lidx_hbm, r_lidx_hbm, u_idx_hbm, r_idx_hbm, u_lin_hbm, r_lin_hbm,
          u_tail_hbm, r_tail_hbm, u_bias_hbm, r_bias_hbm, partial_hbm, ubrb_hbm,
          lidx_u, lidx_r, idx_u, idx_r, u_vals, r_vals, ut_v, rt_v, ub_v, rb_v,
          ubrb_v, acc_v, sem):
        wid = lax.axis_index("s") * NC + lax.axis_index("c")
        base = pl.multiple_of(wid * BPW, 8)
        row0 = wid * NCH

        pltpu.sync_copy(u_lidx_hbm.at[wid], lidx_u)
        pltpu.sync_copy(r_lidx_hbm.at[wid], lidx_r)
        pltpu.sync_copy(u_idx_hbm.at[pl.ds(row0, NCH)], idx_u)
        pltpu.sync_copy(r_idx_hbm.at[pl.ds(row0, NCH)], idx_r)
        pltpu.sync_copy(u_tail_hbm, ut_v)
        pltpu.sync_copy(r_tail_hbm, rt_v)

        for j in range(NCH):
            sl = pl.ds(j * CH, CH)
            copies = []
            for l in range(E):
                copies.append(pltpu.async_copy(
                    u_lin_hbm.at[lidx_u.at[l, sl]], u_vals.at[l, sl], sem))
                copies.append(pltpu.async_copy(
                    r_lin_hbm.at[lidx_r.at[l, sl]], r_vals.at[l, sl], sem))
            copies.append(pltpu.async_copy(u_bias_hbm.at[idx_u.at[j]], ub_v.at[sl], sem))
            copies.append(pltpu.async_copy(r_bias_hbm.at[idx_r.at[j]], rb_v.at[sl], sem))
            for c in copies:
                c.wait()

        acc = jnp.zeros((E,), jnp.float32)
        for j in range(NCH):
            for i8 in range(CH // E):
                g = pl.ds(j * CH + i8 * E, E)
                gi = pl.ds(i8 * E, E)
                iu = idx_u[j, gi]
                ir = idx_r[j, gi]
                mu = iu >= VA
                mr = ir >= VA
                tu = jnp.maximum(iu - VA, 0)
                tr = jnp.maximum(ir - VA, 0)
                for l in range(E):
                    lcol = jnp.full((E,), l, jnp.int32)
                    uv = jnp.where(mu, plsc.load_gather(ut_v, [tu, lcol]),
                                   u_vals[l, g])
                    rv = jnp.where(mr, plsc.load_gather(rt_v, [tr, lcol]),
                                   r_vals[l, g])
                    acc = acc + uv * rv
        acc_v[...] = acc
        pltpu.sync_copy(acc_v, partial_hbm.at[pl.ds(pl.multiple_of(wid * E, 8), E)])

        for i in range(BPW // E):
            sl = pl.ds(i * E, E)
            ubrb_v[sl] = ub_v[sl] + rb_v[sl]
        pltpu.sync_copy(ubrb_v, ubrb_hbm.at[pl.ds(base, BPW)])

    return k(u_lidx, r_lidx, u_idx2d, r_idx2d, u_lin, r_lin,
             u_tail, r_tail, u_bias, r_bias)


def _tc_finish(partials_2d, ubrb_2d):
    def body(p_ref, x_ref, o_ref):
        s = jnp.sum(p_ref[...])
        o_ref[...] = jax.nn.sigmoid(x_ref[...] + s)

    return pl.pallas_call(
        body,
        out_shape=jax.ShapeDtypeStruct(ubrb_2d.shape, jnp.float32),
    )(partials_2d, ubrb_2d)


def kernel(inputs, user_embedding, user_bias, resto_embedding, resto_bias):
    idx = inputs.astype(jnp.int32)
    u_idx = idx[:, 0]
    r_idx = idx[:, 1]
    lane_off = (jnp.arange(E, dtype=jnp.int32) * PAD).reshape(1, E, 1)
    u_lidx = jnp.minimum(u_idx, VA - 1).reshape(NW, 1, BPW) + lane_off
    r_lidx = jnp.minimum(r_idx, VA - 1).reshape(NW, 1, BPW) + lane_off
    u_lin, r_lin = _sc_linearize(user_embedding.T, resto_embedding.T)
    partials, ubrb = _sc_gather_dot(
        u_lidx, r_lidx,
        u_idx.reshape(B // CH, CH), r_idx.reshape(B // CH, CH),
        u_lin, r_lin,
        user_embedding[VA:, :], resto_embedding[VA:, :],
        user_bias.reshape(-1), resto_bias.reshape(-1))
    out = _tc_finish(partials.reshape(NW * E // 128, 128), ubrb.reshape(B // 128, 128))
    return out.reshape(B, 1)
